# Initial kernel scaffold; baseline (speedup 1.0000x reference)
import jax
import jax.numpy as jnp
from jax.experimental import pallas as pl


def kernel(user_indices, item_indices, emb_user_mf, emb_item_mf, emb_user_mlp, emb_item_mlp, W1, b1, W2, b2, W3, b3, W_out, b_out):
    raise NotImplementedError("write your pallas kernel here")


# speedup vs baseline: 2.3041x; 2.3041x over previous
import functools

import jax
import jax.numpy as jnp
from jax import lax
from jax.experimental import pallas as pl
from jax.experimental.pallas import tpu as pltpu
from jax.experimental.pallas import tpu_sc as plsc

_B = 16384
_D = 128
_NW = 32
_BPW = _B // _NW
_BBLK = 1024


def _gather_body(uidx_hbm, iidx_hbm, t_umf, t_imf, t_umlp, t_imlp,
                 o_umf, o_imf, o_umlp, o_imlp,
                 uidx_v, iidx_v, rows_v, sem):
    wid = lax.axis_index("s") * 2 + lax.axis_index("c")
    base = wid * _BPW
    pltpu.sync_copy(uidx_hbm.at[pl.ds(base, _BPW)], uidx_v)
    pltpu.sync_copy(iidx_hbm.at[pl.ds(base, _BPW)], iidx_v)
    for tbl, idx, out in ((t_umf, uidx_v, o_umf),
                          (t_imf, iidx_v, o_imf),
                          (t_umlp, uidx_v, o_umlp),
                          (t_imlp, iidx_v, o_imlp)):
        pltpu.async_copy(tbl.at[idx], rows_v, sem).wait()
        pltpu.sync_copy(rows_v, out.at[pl.ds(base, _BPW)])


def _gather_sc(uidx, iidx, t_umf, t_imf, t_umlp, t_imlp):
    mesh = plsc.VectorSubcoreMesh(core_axis_name="c", subcore_axis_name="s")
    row_ty = jax.ShapeDtypeStruct((_B, _D), jnp.float32)
    run = functools.partial(
        pl.kernel, mesh=mesh,
        out_type=[row_ty, row_ty, row_ty, row_ty],
        scratch_types=[
            pltpu.VMEM((_BPW,), jnp.int32),
            pltpu.VMEM((_BPW,), jnp.int32),
            pltpu.VMEM((_BPW, _D), jnp.float32),
            pltpu.SemaphoreType.DMA,
        ],
    )(_gather_body)
    return run(uidx, iidx, t_umf, t_imf, t_umlp, t_imlp)


def _dense_body(u_mlp, i_mlp, u_mf, i_mf, W1, b1, W2, b2, W3, b3, wo, bo,
                out):
    f32 = jnp.float32
    bf16 = jnp.bfloat16
    xu = u_mlp[...].astype(bf16)
    xi = i_mlp[...].astype(bf16)
    W1v = W1[...].astype(bf16)
    h = jnp.dot(xu, W1v[:_D], preferred_element_type=f32)
    h = h + jnp.dot(xi, W1v[_D:], preferred_element_type=f32)
    h = jnp.maximum(h + b1[...], 0.0).astype(bf16)
    h = jnp.dot(h, W2[...].astype(bf16), preferred_element_type=f32)
    h = jnp.maximum(h + b2[...], 0.0).astype(bf16)
    h = jnp.dot(h, W3[...].astype(bf16), preferred_element_type=f32)
    h = jnp.maximum(h + b3[...], 0.0)
    mf = u_mf[...] * i_mf[...]
    wo_v = wo[...]
    acc = jnp.sum(mf * wo_v[:, :_D], axis=1, keepdims=True)
    acc = acc + jnp.sum(h * wo_v[:, _D:], axis=1, keepdims=True)
    out[...] = jax.nn.sigmoid(acc + bo[...])


def _dense_tc(u_mlp_g, i_mlp_g, u_mf_g, i_mf_g, W1, b1, W2, b2, W3, b3,
              W_out, b_out):
    b1v = b1.reshape(1, -1)
    b2v = b2.reshape(1, -1)
    b3v = b3.reshape(1, -1)
    wov = W_out.reshape(1, -1)
    bov = b_out.reshape(1, 1)
    blk = lambda r, c: pl.BlockSpec((r, c), lambda i: (0, 0))
    act = pl.BlockSpec((_BBLK, _D), lambda i: (i, 0))
    return pl.pallas_call(
        _dense_body,
        grid=(_B // _BBLK,),
        in_specs=[act, act, act, act,
                  blk(256, 1024), blk(1, 1024),
                  blk(1024, 512), blk(1, 512),
                  blk(512, 128), blk(1, 128),
                  blk(1, 256), blk(1, 1)],
        out_specs=pl.BlockSpec((_BBLK, 1), lambda i: (i, 0)),
        out_shape=jax.ShapeDtypeStruct((_B, 1), jnp.float32),
        compiler_params=pltpu.CompilerParams(
            dimension_semantics=("arbitrary",)),
    )(u_mlp_g, i_mlp_g, u_mf_g, i_mf_g, W1, b1v, W2, b2v, W3, b3v, wov, bov)


def kernel(user_indices, item_indices, emb_user_mf, emb_item_mf,
           emb_user_mlp, emb_item_mlp, W1, b1, W2, b2, W3, b3, W_out, b_out):
    uidx = user_indices.astype(jnp.int32)
    iidx = item_indices.astype(jnp.int32)
    o_umf, o_imf, o_umlp, o_imlp = _gather_sc(
        uidx, iidx, emb_user_mf, emb_item_mf, emb_user_mlp, emb_item_mlp)
    return _dense_tc(o_umlp, o_imlp, o_umf, o_imf,
                     W1, b1, W2, b2, W3, b3, W_out, b_out)


# speedup vs baseline: 2.4868x; 1.0793x over previous
import functools

import jax
import jax.numpy as jnp
from jax import lax
from jax.experimental import pallas as pl
from jax.experimental.pallas import tpu as pltpu
from jax.experimental.pallas import tpu_sc as plsc

_B = 16384
_D = 128
_NW = 32
_BPW = _B // _NW
_CH = 256
_NCH = _BPW // _CH
_BBLK = 1024


def _gather_body(uidx_hbm, iidx_hbm, t_umf, t_imf, t_umlp, t_imlp,
                 o_x, o_umf, o_imf,
                 uidx_v, iidx_v, buf0, buf1, gsem0, gsem1, ssem0, ssem1):
    wid = lax.axis_index("s") * 2 + lax.axis_index("c")
    base = wid * _BPW
    pltpu.sync_copy(uidx_hbm.at[pl.ds(base, _BPW)], uidx_v)
    pltpu.sync_copy(iidx_hbm.at[pl.ds(base, _BPW)], iidx_v)
    bufs = (buf0, buf1)
    gsems = (gsem0, gsem1)
    ssems = (ssem0, ssem1)
    tasks = []
    for c in range(_NCH):
        r0 = c * _CH
        for tbl, idxv, out, col in ((t_umlp, uidx_v, o_x, 0),
                                    (t_imlp, iidx_v, o_x, _D),
                                    (t_umf, uidx_v, o_umf, None),
                                    (t_imf, iidx_v, o_imf, None)):
            tasks.append((tbl, idxv, r0, out, col))
    scat = [None, None]
    for t, (tbl, idxv, r0, out, col) in enumerate(tasks):
        b = t % 2
        if scat[b] is not None:
            scat[b].wait()
        pltpu.async_copy(tbl.at[idxv.at[pl.ds(r0, _CH)]], bufs[b],
                         gsems[b]).wait()
        if col is None:
            dst = out.at[pl.ds(base + r0, _CH)]
        else:
            dst = out.at[pl.ds(base + r0, _CH), pl.ds(col, _D)]
        scat[b] = pltpu.async_copy(bufs[b], dst, ssems[b])
    scat[0].wait()
    scat[1].wait()


def _gather_sc(uidx, iidx, t_umf, t_imf, t_umlp, t_imlp):
    mesh = plsc.VectorSubcoreMesh(core_axis_name="c", subcore_axis_name="s")
    run = functools.partial(
        pl.kernel, mesh=mesh,
        out_type=[jax.ShapeDtypeStruct((_B, 2 * _D), jnp.float32),
                  jax.ShapeDtypeStruct((_B, _D), jnp.float32),
                  jax.ShapeDtypeStruct((_B, _D), jnp.float32)],
        scratch_types=[
            pltpu.VMEM((_BPW,), jnp.int32),
            pltpu.VMEM((_BPW,), jnp.int32),
            pltpu.VMEM((_CH, _D), jnp.float32),
            pltpu.VMEM((_CH, _D), jnp.float32),
            pltpu.SemaphoreType.DMA,
            pltpu.SemaphoreType.DMA,
            pltpu.SemaphoreType.DMA,
            pltpu.SemaphoreType.DMA,
        ],
    )(_gather_body)
    return run(uidx, iidx, t_umf, t_imf, t_umlp, t_imlp)


def _dense_body(x, u_mf, i_mf, W1, b1, W2, b2, W3, b3, wo, bo, out):
    f32 = jnp.float32
    bf16 = jnp.bfloat16
    h = jnp.dot(x[...].astype(bf16), W1[...].astype(bf16),
                preferred_element_type=f32)
    h = jnp.maximum(h + b1[...], 0.0).astype(bf16)
    h = jnp.dot(h, W2[...].astype(bf16), preferred_element_type=f32)
    h = jnp.maximum(h + b2[...], 0.0).astype(bf16)
    h = jnp.dot(h, W3[...].astype(bf16), preferred_element_type=f32)
    h = jnp.maximum(h + b3[...], 0.0)
    mf = u_mf[...] * i_mf[...]
    wo_v = wo[...]
    acc = jnp.sum(mf * wo_v[:, :_D], axis=1, keepdims=True)
    acc = acc + jnp.sum(h * wo_v[:, _D:], axis=1, keepdims=True)
    out[...] = jax.nn.sigmoid(acc + bo[...])


def _dense_tc(x_g, u_mf_g, i_mf_g, W1, b1, W2, b2, W3, b3, W_out, b_out):
    b1v = b1.reshape(1, -1)
    b2v = b2.reshape(1, -1)
    b3v = b3.reshape(1, -1)
    wov = W_out.reshape(1, -1)
    bov = b_out.reshape(1, 1)
    blk = lambda r, c: pl.BlockSpec((r, c), lambda i: (0, 0))
    act = pl.BlockSpec((_BBLK, _D), lambda i: (i, 0))
    return pl.pallas_call(
        _dense_body,
        grid=(_B // _BBLK,),
        in_specs=[pl.BlockSpec((_BBLK, 2 * _D), lambda i: (i, 0)), act, act,
                  blk(256, 1024), blk(1, 1024),
                  blk(1024, 512), blk(1, 512),
                  blk(512, 128), blk(1, 128),
                  blk(1, 256), blk(1, 1)],
        out_specs=pl.BlockSpec((_BBLK, 1), lambda i: (i, 0)),
        out_shape=jax.ShapeDtypeStruct((_B, 1), jnp.float32),
        compiler_params=pltpu.CompilerParams(
            dimension_semantics=("arbitrary",)),
    )(x_g, u_mf_g, i_mf_g, W1, b1v, W2, b2v, W3, b3v, wov, bov)


def kernel(user_indices, item_indices, emb_user_mf, emb_item_mf,
           emb_user_mlp, emb_item_mlp, W1, b1, W2, b2, W3, b3, W_out, b_out):
    uidx = user_indices.astype(jnp.int32)
    iidx = item_indices.astype(jnp.int32)
    x_g, o_umf, o_imf = _gather_sc(
        uidx, iidx, emb_user_mf, emb_item_mf, emb_user_mlp, emb_item_mlp)
    return _dense_tc(x_g, o_umf, o_imf, W1, b1, W2, b2, W3, b3, W_out, b_out)


# speedup vs baseline: 2.6095x; 1.0493x over previous
import functools

import jax
import jax.numpy as jnp
from jax import lax
from jax.experimental import pallas as pl
from jax.experimental.pallas import tpu as pltpu
from jax.experimental.pallas import tpu_sc as plsc

_B = 16384
_D = 128
_NW = 32
_NSPLIT = 2
_BS = _B // _NSPLIT
_BPW = _BS // _NW
_CH = 256
_NCH = _BPW // _CH
_BBLK = 1024


def _gather_body(uidx_hbm, iidx_hbm, t_umf, t_imf, t_umlp, t_imlp,
                 o_x, o_umf, o_imf,
                 uidx_v, iidx_v, buf0, buf1, gsem0, gsem1, ssem0, ssem1):
    wid = lax.axis_index("s") * 2 + lax.axis_index("c")
    base = wid * _BPW
    pltpu.sync_copy(uidx_hbm.at[pl.ds(base, _BPW)], uidx_v)
    pltpu.sync_copy(iidx_hbm.at[pl.ds(base, _BPW)], iidx_v)
    bufs = (buf0, buf1)
    gsems = (gsem0, gsem1)
    ssems = (ssem0, ssem1)
    tasks = []
    for c in range(_NCH):
        r0 = c * _CH
        for tbl, idxv, out, col in ((t_umlp, uidx_v, o_x, 0),
                                    (t_imlp, iidx_v, o_x, _D),
                                    (t_umf, uidx_v, o_umf, None),
                                    (t_imf, iidx_v, o_imf, None)):
            tasks.append((tbl, idxv, r0, out, col))
    scat = [None, None]
    for t, (tbl, idxv, r0, out, col) in enumerate(tasks):
        b = t % 2
        if scat[b] is not None:
            scat[b].wait()
        pltpu.async_copy(tbl.at[idxv.at[pl.ds(r0, _CH)]], bufs[b],
                         gsems[b]).wait()
        if col is None:
            dst = out.at[pl.ds(base + r0, _CH)]
        else:
            dst = out.at[pl.ds(base + r0, _CH), pl.ds(col, _D)]
        scat[b] = pltpu.async_copy(bufs[b], dst, ssems[b])
    scat[0].wait()
    scat[1].wait()


def _gather_sc(uidx, iidx, t_umf, t_imf, t_umlp, t_imlp):
    mesh = plsc.VectorSubcoreMesh(core_axis_name="c", subcore_axis_name="s")
    run = functools.partial(
        pl.kernel, mesh=mesh,
        out_type=[jax.ShapeDtypeStruct((_BS, 2 * _D), jnp.float32),
                  jax.ShapeDtypeStruct((_BS, _D), jnp.float32),
                  jax.ShapeDtypeStruct((_BS, _D), jnp.float32)],
        scratch_types=[
            pltpu.VMEM((_BPW,), jnp.int32),
            pltpu.VMEM((_BPW,), jnp.int32),
            pltpu.VMEM((_CH, _D), jnp.float32),
            pltpu.VMEM((_CH, _D), jnp.float32),
            pltpu.SemaphoreType.DMA,
            pltpu.SemaphoreType.DMA,
            pltpu.SemaphoreType.DMA,
            pltpu.SemaphoreType.DMA,
        ],
    )(_gather_body)
    return run(uidx, iidx, t_umf, t_imf, t_umlp, t_imlp)


def _dense_body(x, u_mf, i_mf, W1, b1, W2, b2, W3, b3, wo, bo, out):
    f32 = jnp.float32
    bf16 = jnp.bfloat16
    h = jnp.dot(x[...].astype(bf16), W1[...].astype(bf16),
                preferred_element_type=f32)
    h = jnp.maximum(h + b1[...], 0.0).astype(bf16)
    h = jnp.dot(h, W2[...].astype(bf16), preferred_element_type=f32)
    h = jnp.maximum(h + b2[...], 0.0).astype(bf16)
    h = jnp.dot(h, W3[...].astype(bf16), preferred_element_type=f32)
    h = jnp.maximum(h + b3[...], 0.0)
    mf = u_mf[...] * i_mf[...]
    wo_v = wo[...]
    acc = jnp.sum(mf * wo_v[:, :_D], axis=1, keepdims=True)
    acc = acc + jnp.sum(h * wo_v[:, _D:], axis=1, keepdims=True)
    out[...] = jax.nn.sigmoid(acc + bo[...])


def _dense_tc(x_g, u_mf_g, i_mf_g, W1, b1, W2, b2, W3, b3, W_out, b_out):
    b1v = b1.reshape(1, -1)
    b2v = b2.reshape(1, -1)
    b3v = b3.reshape(1, -1)
    wov = W_out.reshape(1, -1)
    bov = b_out.reshape(1, 1)
    blk = lambda r, c: pl.BlockSpec((r, c), lambda i: (0, 0))
    act = pl.BlockSpec((_BBLK, _D), lambda i: (i, 0))
    return pl.pallas_call(
        _dense_body,
        grid=(_BS // _BBLK,),
        in_specs=[pl.BlockSpec((_BBLK, 2 * _D), lambda i: (i, 0)), act, act,
                  blk(256, 1024), blk(1, 1024),
                  blk(1024, 512), blk(1, 512),
                  blk(512, 128), blk(1, 128),
                  blk(1, 256), blk(1, 1)],
        out_specs=pl.BlockSpec((_BBLK, 1), lambda i: (i, 0)),
        out_shape=jax.ShapeDtypeStruct((_BS, 1), jnp.float32),
        compiler_params=pltpu.CompilerParams(
            dimension_semantics=("arbitrary",)),
    )(x_g, u_mf_g, i_mf_g, W1, b1v, W2, b2v, W3, b3v, wov, bov)


def kernel(user_indices, item_indices, emb_user_mf, emb_item_mf,
           emb_user_mlp, emb_item_mlp, W1, b1, W2, b2, W3, b3, W_out, b_out):
    uidx = user_indices.astype(jnp.int32)
    iidx = item_indices.astype(jnp.int32)
    outs = []
    for s in range(_NSPLIT):
        lo = s * _BS
        x_g, o_umf, o_imf = _gather_sc(
            lax.dynamic_slice_in_dim(uidx, lo, _BS),
            lax.dynamic_slice_in_dim(iidx, lo, _BS),
            emb_user_mf, emb_item_mf, emb_user_mlp, emb_item_mlp)
        outs.append(_dense_tc(x_g, o_umf, o_imf,
                              W1, b1, W2, b2, W3, b3, W_out, b_out))
    return jnp.concatenate(outs, axis=0)
